# baseline (speedup 1.0000x reference)
import functools

import jax
import jax.numpy as jnp
from jax import lax
from jax.experimental import pallas as pl
from jax.experimental.pallas import tpu as pltpu
from jax.experimental.pallas import tpu_sc as plsc

B = 16384
D = 64
NE = 1_000_000
S0 = 256_000
L = 16
NC, NS = 2, 16
NW = NC * NS
BPW = B // NW
HB = 256
CH = 128

EB = 6400
NBLK = S0 // EB

_mesh = plsc.VectorSubcoreMesh(core_axis_name="c", subcore_axis_name="s")


def _pack_body(s0_ref, s1_ref, s2_ref, s3_ref, out_ref):
    pieces = []
    for ref in (s0_ref, s1_ref, s2_ref, s3_ref):
        bits = lax.bitcast_convert_type(ref[...], jnp.uint32)
        lo = bits[0:32, :] >> 16
        hi = bits[32:64, :] & jnp.uint32(0xFFFF0000)
        pieces.append(lax.bitcast_convert_type(hi | lo, jnp.int32))
    out_ref[...] = jnp.concatenate(pieces, axis=0).T


@jax.jit
def _pack(ent_t):
    last_blk = NE // EB
    return pl.pallas_call(
        _pack_body,
        grid=(NBLK,),
        in_specs=[
            pl.BlockSpec((D, EB),
                         functools.partial(
                             lambda s, i: (0, jnp.minimum(s * NBLK + i,
                                                          last_blk)), s))
            for s in range(4)
        ],
        out_specs=pl.BlockSpec((EB, 128), lambda i: (i, 0)),
        out_shape=jax.ShapeDtypeStruct((S0, 128), jnp.int32),
    )(ent_t, ent_t, ent_t, ent_t)


def _neg_sqrt(x):
    i = lax.bitcast_convert_type(x, jnp.int32)
    r = lax.bitcast_convert_type(jnp.int32(0x5F3759DF) - (i >> 1), jnp.float32)
    for _ in range(3):
        r = r * (1.5 - 0.5 * x * r * r)
    return -(x * r)


def _unpack2(v):
    lo = lax.bitcast_convert_type(v << 16, jnp.float32)
    hi = lax.bitcast_convert_type(v & jnp.int32(-65536), jnp.float32)
    return lo, hi


def _rowseg(e):
    r = jnp.where(e >= 2 * S0, e - 2 * S0, e)
    r = jnp.where(r >= S0, r - S0, r)
    seg = ((e >= S0).astype(jnp.int32) + (e >= 2 * S0).astype(jnp.int32)
           + (e >= 3 * S0).astype(jnp.int32))
    return r, seg << 5


@functools.partial(
    pl.kernel,
    out_type=jax.ShapeDtypeStruct((B,), jnp.float32),
    mesh=_mesh,
    compiler_params=pltpu.CompilerParams(needs_layout_passes=False,
                                         use_tc_tiling_on_sc=True),
    scratch_types=[
        pltpu.VMEM((BPW,), jnp.int32),
        pltpu.VMEM((BPW,), jnp.int32),
        pltpu.VMEM((BPW,), jnp.int32),
        pltpu.VMEM((BPW,), jnp.int32),
        pltpu.VMEM((BPW,), jnp.int32),
        pltpu.VMEM((BPW,), jnp.int32),
        pltpu.VMEM((2 * CH, 128), jnp.int32),
        pltpu.VMEM((2 * CH, 128), jnp.int32),
        pltpu.VMEM((2 * CH, 128), jnp.int32),
        pltpu.VMEM((BPW,), jnp.float32),
        pltpu.SemaphoreType.DMA,
        pltpu.SemaphoreType.DMA,
    ],
)
def _sc_score(head_hbm, rel_hbm, tail_hbm, ent2_hbm, rel2_hbm, out_hbm,
              hraw, rraw, traw, hpair, rpair, tpair, hrow, rrow, trow,
              outv, sem0, sem1):
    wid = lax.axis_index("s") * NC + lax.axis_index("c")
    base = pl.multiple_of(wid * BPW, BPW)

    pltpu.sync_copy(head_hbm.at[pl.ds(base, BPW)], hraw)
    pltpu.sync_copy(rel_hbm.at[pl.ds(base, BPW)], rraw)
    pltpu.sync_copy(tail_hbm.at[pl.ds(base, BPW)], traw)

    def pair_body(i, carry):
        sl = pl.ds(pl.multiple_of(i * L, L), L)
        hpair[sl] = _rowseg(hraw[sl])[0]
        tpair[sl] = _rowseg(traw[sl])[0]
        rv = rraw[sl]
        rr = jnp.where(rv >= 500, rv - 500, rv)
        rpair[sl] = jnp.where(rr >= 250, rr - 250, rr)
        return carry

    lax.fori_loop(0, BPW // L, pair_body, 0)

    NQ = BPW // CH
    sems = (sem0, sem1)

    def fire(q):
        isl = pl.ds(pl.multiple_of(q * CH, CH), CH)
        dsl = pl.ds((q % 2) * CH, CH)
        s = sems[q % 2]
        return (pltpu.async_copy(ent2_hbm.at[hpair.at[isl]], hrow.at[dsl], s),
                pltpu.async_copy(rel2_hbm.at[rpair.at[isl]], rrow.at[dsl], s),
                pltpu.async_copy(ent2_hbm.at[tpair.at[isl]], trow.at[dsl], s))

    pend = fire(0)
    for q in range(NQ):
        for c in pend:
            c.wait()
        if q + 1 < NQ:
            pend = fire(q + 1)

        qoff = (q % 2) * CH

        def group_body(g, carry):
            rows = lax.iota(jnp.int32, L) + (g * L + qoff)
            gsl = pl.ds(pl.multiple_of(q * CH + g * L, L), L)
            hsel = _rowseg(hraw[gsl])[1]
            tsel = _rowseg(traw[gsl])[1]
            rv = rraw[gsl]
            rsel = (((rv >= 250).astype(jnp.int32)
                     + (rv >= 500).astype(jnp.int32)
                     + (rv >= 750).astype(jnp.int32)) << 5)

            def col_body(i, acc):
                for dd in range(2):
                    d = i * 2 + dd
                    hlo, hhi = _unpack2(
                        plsc.load_gather(hrow, [rows, hsel + d]))
                    rlo, rhi = _unpack2(
                        plsc.load_gather(rrow, [rows, rsel + d]))
                    tlo, thi = _unpack2(
                        plsc.load_gather(trow, [rows, tsel + d]))
                    elo = hlo + rlo - tlo
                    ehi = hhi + rhi - thi
                    acc = acc + elo * elo + ehi * ehi
                return acc

            x = lax.fori_loop(0, D // 4, col_body,
                              jnp.zeros((L,), jnp.float32)) + 1e-12
            outv[gsl] = _neg_sqrt(x)
            return carry

        lax.fori_loop(0, CH // L, group_body, 0)

    pltpu.sync_copy(outv, out_hbm.at[pl.ds(base, BPW)])


def _pack_rel(rel):
    bits = lax.bitcast_convert_type(rel, jnp.uint32)
    packed = (bits[:, 32:64] & jnp.uint32(0xFFFF0000)) | (bits[:, 0:32] >> 16)
    packed = lax.bitcast_convert_type(packed, jnp.int32)
    return packed.reshape(4, 250, 32).transpose(1, 0, 2).reshape(250, 128)


def kernel(head, relation, tail, entity_embeddings, relation_embeddings):
    ent2 = _pack(entity_embeddings.T)
    rel2 = _pack_rel(relation_embeddings)
    return _sc_score(head.astype(jnp.int32), relation.astype(jnp.int32),
                     tail.astype(jnp.int32), ent2, rel2)


# speedup vs baseline: 1.0203x; 1.0203x over previous
import functools

import jax
import jax.numpy as jnp
from jax import lax
from jax.experimental import pallas as pl
from jax.experimental.pallas import tpu as pltpu
from jax.experimental.pallas import tpu_sc as plsc

B = 16384
D = 64
NE = 1_000_000
S0 = 256_000
L = 16
NC, NS = 2, 16
NW = NC * NS
BPW = B // NW
HB = 256
CH = 128
QC = 64

EB = 10240
NBLK = S0 // EB

_mesh = plsc.VectorSubcoreMesh(core_axis_name="c", subcore_axis_name="s")


def _pack_body(s0_ref, s1_ref, s2_ref, s3_ref, out_ref):
    pieces = []
    for ref in (s0_ref, s1_ref, s2_ref, s3_ref):
        bits = lax.bitcast_convert_type(ref[...], jnp.uint32)
        lo = bits[0:32, :] >> 16
        hi = bits[32:64, :] & jnp.uint32(0xFFFF0000)
        pieces.append(lax.bitcast_convert_type(hi | lo, jnp.int32))
    out_ref[...] = jnp.concatenate(pieces, axis=0).T


@jax.jit
def _pack(ent_t):
    last_blk = NE // EB
    return pl.pallas_call(
        _pack_body,
        grid=(NBLK,),
        in_specs=[
            pl.BlockSpec((D, EB),
                         functools.partial(
                             lambda s, i: (0, jnp.minimum(s * NBLK + i,
                                                          last_blk)), s))
            for s in range(4)
        ],
        out_specs=pl.BlockSpec((EB, 128), lambda i: (i, 0)),
        out_shape=jax.ShapeDtypeStruct((S0, 128), jnp.int32),
    )(ent_t, ent_t, ent_t, ent_t)


def _neg_sqrt(x):
    i = lax.bitcast_convert_type(x, jnp.int32)
    r = lax.bitcast_convert_type(jnp.int32(0x5F3759DF) - (i >> 1), jnp.float32)
    for _ in range(3):
        r = r * (1.5 - 0.5 * x * r * r)
    return -(x * r)


def _unpack2(v):
    lo = lax.bitcast_convert_type(v << 16, jnp.float32)
    hi = lax.bitcast_convert_type(v & jnp.int32(-65536), jnp.float32)
    return lo, hi


def _rowseg(e):
    r = jnp.where(e >= 2 * S0, e - 2 * S0, e)
    r = jnp.where(r >= S0, r - S0, r)
    seg = ((e >= S0).astype(jnp.int32) + (e >= 2 * S0).astype(jnp.int32)
           + (e >= 3 * S0).astype(jnp.int32))
    return r, seg << 5


@functools.partial(
    pl.kernel,
    out_type=jax.ShapeDtypeStruct((B,), jnp.float32),
    mesh=_mesh,
    compiler_params=pltpu.CompilerParams(needs_layout_passes=False,
                                         use_tc_tiling_on_sc=True),
    scratch_types=[
        pltpu.VMEM((BPW,), jnp.int32),
        pltpu.VMEM((BPW,), jnp.int32),
        pltpu.VMEM((BPW,), jnp.int32),
        pltpu.VMEM((BPW,), jnp.int32),
        pltpu.VMEM((BPW,), jnp.int32),
        pltpu.VMEM((BPW,), jnp.int32),
        pltpu.VMEM((2 * QC, 128), jnp.int32),
        pltpu.VMEM((2 * QC, 128), jnp.int32),
        pltpu.VMEM((2 * QC, 128), jnp.int32),
        pltpu.VMEM((BPW,), jnp.float32),
        pltpu.SemaphoreType.DMA,
        pltpu.SemaphoreType.DMA,
    ],
)
def _sc_score(head_hbm, rel_hbm, tail_hbm, ent2_hbm, rel2_hbm, out_hbm,
              hraw, rraw, traw, hpair, rpair, tpair, hrow, rrow, trow,
              outv, sem0, sem1):
    wid = lax.axis_index("s") * NC + lax.axis_index("c")
    base = pl.multiple_of(wid * BPW, BPW)

    pltpu.sync_copy(head_hbm.at[pl.ds(base, BPW)], hraw)
    pltpu.sync_copy(rel_hbm.at[pl.ds(base, BPW)], rraw)
    pltpu.sync_copy(tail_hbm.at[pl.ds(base, BPW)], traw)

    def pair_body(i, carry):
        sl = pl.ds(pl.multiple_of(i * L, L), L)
        hpair[sl] = _rowseg(hraw[sl])[0]
        tpair[sl] = _rowseg(traw[sl])[0]
        rv = rraw[sl]
        rr = jnp.where(rv >= 500, rv - 500, rv)
        rpair[sl] = jnp.where(rr >= 250, rr - 250, rr)
        return carry

    lax.fori_loop(0, BPW // L, pair_body, 0)

    NQ = BPW // QC
    sems = (sem0, sem1)

    def fire(q):
        isl = pl.ds(pl.multiple_of(q * QC, QC), QC)
        dsl = pl.ds((q % 2) * QC, QC)
        s = sems[q % 2]
        return (pltpu.async_copy(ent2_hbm.at[hpair.at[isl]], hrow.at[dsl], s),
                pltpu.async_copy(rel2_hbm.at[rpair.at[isl]], rrow.at[dsl], s),
                pltpu.async_copy(ent2_hbm.at[tpair.at[isl]], trow.at[dsl], s))

    pend = fire(0)
    for q in range(NQ):
        for c in pend:
            c.wait()
        if q + 1 < NQ:
            pend = fire(q + 1)

        qoff = (q % 2) * QC

        def group_body(g, carry):
            rows = lax.iota(jnp.int32, L) + (g * L + qoff)
            gsl = pl.ds(pl.multiple_of(q * QC + g * L, L), L)
            hsel = _rowseg(hraw[gsl])[1]
            tsel = _rowseg(traw[gsl])[1]
            rv = rraw[gsl]
            rsel = (((rv >= 250).astype(jnp.int32)
                     + (rv >= 500).astype(jnp.int32)
                     + (rv >= 750).astype(jnp.int32)) << 5)

            def col_body(i, acc):
                for dd in range(2):
                    d = i * 2 + dd
                    hlo, hhi = _unpack2(
                        plsc.load_gather(hrow, [rows, hsel + d]))
                    rlo, rhi = _unpack2(
                        plsc.load_gather(rrow, [rows, rsel + d]))
                    tlo, thi = _unpack2(
                        plsc.load_gather(trow, [rows, tsel + d]))
                    elo = hlo + rlo - tlo
                    ehi = hhi + rhi - thi
                    acc = acc + elo * elo + ehi * ehi
                return acc

            x = lax.fori_loop(0, D // 4, col_body,
                              jnp.zeros((L,), jnp.float32)) + 1e-12
            outv[gsl] = _neg_sqrt(x)
            return carry

        lax.fori_loop(0, QC // L, group_body, 0)

    pltpu.sync_copy(outv, out_hbm.at[pl.ds(base, BPW)])


def _pack_rel(rel):
    bits = lax.bitcast_convert_type(rel, jnp.uint32)
    packed = (bits[:, 32:64] & jnp.uint32(0xFFFF0000)) | (bits[:, 0:32] >> 16)
    packed = lax.bitcast_convert_type(packed, jnp.int32)
    return packed.reshape(4, 250, 32).transpose(1, 0, 2).reshape(250, 128)


def kernel(head, relation, tail, entity_embeddings, relation_embeddings):
    ent2 = _pack(entity_embeddings.T)
    rel2 = _pack_rel(relation_embeddings)
    return _sc_score(head.astype(jnp.int32), relation.astype(jnp.int32),
                     tail.astype(jnp.int32), ent2, rel2)


# speedup vs baseline: 1.0206x; 1.0003x over previous
import functools

import jax
import jax.numpy as jnp
from jax import lax
from jax.experimental import pallas as pl
from jax.experimental.pallas import tpu as pltpu
from jax.experimental.pallas import tpu_sc as plsc

B = 16384
D = 64
NE = 1_000_000
S0 = 256_000
L = 16
NC, NS = 2, 16
NW = NC * NS
BPW = B // NW
HB = 256
CH = 128
QC = 64

EB = 10240
NBLK = S0 // EB

_mesh = plsc.VectorSubcoreMesh(core_axis_name="c", subcore_axis_name="s")


def _pack_body(s0_ref, s1_ref, s2_ref, s3_ref, out_ref):
    pieces = []
    for ref in (s0_ref, s1_ref, s2_ref, s3_ref):
        bits = lax.bitcast_convert_type(ref[...], jnp.uint32)
        lo = bits[0:32, :] >> 16
        hi = bits[32:64, :] & jnp.uint32(0xFFFF0000)
        pieces.append(lax.bitcast_convert_type(hi | lo, jnp.int32))
    out_ref[...] = jnp.concatenate(pieces, axis=0).T


@jax.jit
def _pack(ent_t):
    last_blk = NE // EB
    return pl.pallas_call(
        _pack_body,
        grid=(NBLK,),
        in_specs=[
            pl.BlockSpec((D, EB),
                         functools.partial(
                             lambda s, i: (0, jnp.minimum(s * NBLK + i,
                                                          last_blk)), s))
            for s in range(4)
        ],
        out_specs=pl.BlockSpec((EB, 128), lambda i: (i, 0)),
        out_shape=jax.ShapeDtypeStruct((S0, 128), jnp.int32),
    )(ent_t, ent_t, ent_t, ent_t)


def _neg_sqrt(x):
    i = lax.bitcast_convert_type(x, jnp.int32)
    r = lax.bitcast_convert_type(jnp.int32(0x5F3759DF) - (i >> 1), jnp.float32)
    for _ in range(3):
        r = r * (1.5 - 0.5 * x * r * r)
    return -(x * r)


def _unpack2(v):
    lo = lax.bitcast_convert_type(v << 16, jnp.float32)
    hi = lax.bitcast_convert_type(v & jnp.int32(-65536), jnp.float32)
    return lo, hi


def _rowseg(e):
    r = jnp.where(e >= 2 * S0, e - 2 * S0, e)
    r = jnp.where(r >= S0, r - S0, r)
    seg = ((e >= S0).astype(jnp.int32) + (e >= 2 * S0).astype(jnp.int32)
           + (e >= 3 * S0).astype(jnp.int32))
    return r, seg << 5


@functools.partial(
    pl.kernel,
    out_type=jax.ShapeDtypeStruct((B,), jnp.float32),
    mesh=_mesh,
    compiler_params=pltpu.CompilerParams(needs_layout_passes=False,
                                         use_tc_tiling_on_sc=True),
    scratch_types=[
        pltpu.VMEM((BPW,), jnp.int32),
        pltpu.VMEM((BPW,), jnp.int32),
        pltpu.VMEM((BPW,), jnp.int32),
        pltpu.VMEM((BPW,), jnp.int32),
        pltpu.VMEM((BPW,), jnp.int32),
        pltpu.VMEM((BPW,), jnp.int32),
        pltpu.VMEM((2 * QC, 128), jnp.int32),
        pltpu.VMEM((2 * QC, 128), jnp.int32),
        pltpu.VMEM((2 * QC, 128), jnp.int32),
        pltpu.VMEM((BPW,), jnp.float32),
        pltpu.SemaphoreType.DMA,
        pltpu.SemaphoreType.DMA,
    ],
)
def _sc_score(head_hbm, rel_hbm, tail_hbm, ent2_hbm, rel2_hbm, out_hbm,
              hraw, rraw, traw, hpair, rpair, tpair, hrow, rrow, trow,
              outv, sem0, sem1):
    wid = lax.axis_index("s") * NC + lax.axis_index("c")
    base = pl.multiple_of(wid * BPW, BPW)

    pltpu.sync_copy(head_hbm.at[pl.ds(base, BPW)], hraw)
    pltpu.sync_copy(rel_hbm.at[pl.ds(base, BPW)], rraw)
    pltpu.sync_copy(tail_hbm.at[pl.ds(base, BPW)], traw)

    def pair_body(i, carry):
        sl = pl.ds(pl.multiple_of(i * L, L), L)
        hpair[sl] = _rowseg(hraw[sl])[0]
        tpair[sl] = _rowseg(traw[sl])[0]
        rpair[sl] = rraw[sl] >> 2
        return carry

    lax.fori_loop(0, BPW // L, pair_body, 0)

    NQ = BPW // QC
    sems = (sem0, sem1)

    def fire(q):
        isl = pl.ds(pl.multiple_of(q * QC, QC), QC)
        dsl = pl.ds((q % 2) * QC, QC)
        s = sems[q % 2]
        return (pltpu.async_copy(ent2_hbm.at[hpair.at[isl]], hrow.at[dsl], s),
                pltpu.async_copy(rel2_hbm.at[rpair.at[isl]], rrow.at[dsl], s),
                pltpu.async_copy(ent2_hbm.at[tpair.at[isl]], trow.at[dsl], s))

    pend = fire(0)
    for q in range(NQ):
        for c in pend:
            c.wait()
        if q + 1 < NQ:
            pend = fire(q + 1)

        qoff = (q % 2) * QC

        def group_body(g, carry):
            rows = lax.iota(jnp.int32, L) + (g * L + qoff)
            gsl = pl.ds(pl.multiple_of(q * QC + g * L, L), L)
            hsel = _rowseg(hraw[gsl])[1]
            tsel = _rowseg(traw[gsl])[1]
            rsel = (rraw[gsl] & 3) << 5

            def col_body(i, acc):
                for dd in range(2):
                    d = i * 2 + dd
                    hlo, hhi = _unpack2(
                        plsc.load_gather(hrow, [rows, hsel + d]))
                    rlo, rhi = _unpack2(
                        plsc.load_gather(rrow, [rows, rsel + d]))
                    tlo, thi = _unpack2(
                        plsc.load_gather(trow, [rows, tsel + d]))
                    elo = hlo + rlo - tlo
                    ehi = hhi + rhi - thi
                    acc = acc + elo * elo + ehi * ehi
                return acc

            x = lax.fori_loop(0, D // 4, col_body,
                              jnp.zeros((L,), jnp.float32)) + 1e-12
            outv[gsl] = _neg_sqrt(x)
            return carry

        lax.fori_loop(0, QC // L, group_body, 0)

    pltpu.sync_copy(outv, out_hbm.at[pl.ds(base, BPW)])


def _pack_rel(rel):
    bits = lax.bitcast_convert_type(rel, jnp.uint32)
    packed = (bits[:, 32:64] & jnp.uint32(0xFFFF0000)) | (bits[:, 0:32] >> 16)
    packed = lax.bitcast_convert_type(packed, jnp.int32)
    return packed.reshape(250, 128)


def kernel(head, relation, tail, entity_embeddings, relation_embeddings):
    ent2 = _pack(entity_embeddings.T)
    rel2 = _pack_rel(relation_embeddings)
    return _sc_score(head.astype(jnp.int32), relation.astype(jnp.int32),
                     tail.astype(jnp.int32), ent2, rel2)
